# Initial kernel scaffold; baseline (speedup 1.0000x reference)
import jax
import jax.numpy as jnp
from jax.experimental import pallas as pl


def kernel(x, edge_index, edge_attr, params):
    raise NotImplementedError("write your pallas kernel here")


# speedup vs baseline: 209.6953x; 209.6953x over previous
import functools

import jax
import jax.numpy as jnp
from jax.experimental import pallas as pl

D, H, C, N_OPT = 64, 4, 16, 5
BG = 8


def _body(xcol_ref, ea_ref, s_ref, dm_ref, dmt_ref, g4t_ref, rsel_ref,
          attr_ref, opt_ref, prior_ref,
          lw_l_ref, lb_l_ref, lw_r_ref, lb_r_ref, att_ref, wedge_ref,
          bias_ref, lng_ref, lnb_ref,
          w1_ref, b1_ref, w2_ref, b2_ref,
          hat_ref, rel_ref, fsc_ref, *, n_nodes, n_layers):
    N = n_nodes
    R = BG * N
    xcol = xcol_ref[0]
    ea = ea_ref[...]

    tok = jnp.zeros((R, D), jnp.float32)
    for o in range(N_OPT):
        tok = tok + jnp.where(xcol == o, opt_ref[o], 0.0)
    tok = (attr_ref[...] + tok) * prior_ref[...]

    S = s_ref[...]
    Dm = dm_ref[...]
    DmT = dmt_ref[...]
    G4T = g4t_ref[...]

    h = tok
    for li in range(n_layers):
        xl2 = jnp.dot(h, lw_l_ref[li], preferred_element_type=jnp.float32) + lb_l_ref[li]
        xr2 = jnp.dot(h, lw_r_ref[li], preferred_element_type=jnp.float32) + lb_r_ref[li]
        ee = ea * wedge_ref[li]
        att_row = att_ref[li]
        outs = []
        for g in range(BG):
            xl_g = xl2[g * N:(g + 1) * N]
            xr_g = xr2[g * N:(g + 1) * N]
            xls = jnp.dot(S, xl_g, preferred_element_type=jnp.float32)
            xrd = jnp.dot(Dm, xr_g, preferred_element_type=jnp.float32)
            e = xls + xrd + ee
            e = jnp.where(e >= 0, e, 0.2 * e)
            ew = e * att_row
            alpha = jnp.dot(ew, G4T.T, preferred_element_type=jnp.float32)
            amax = jnp.max(alpha, axis=0, keepdims=True)
            ex = jnp.exp(alpha - amax)
            den = jnp.dot(DmT, ex, preferred_element_type=jnp.float32)
            den_e = jnp.dot(Dm, den, preferred_element_type=jnp.float32)
            a = ex / (den_e + 1e-16)
            a_exp = jnp.dot(a, G4T, preferred_element_type=jnp.float32)
            msg = xls * a_exp
            outs.append(jnp.dot(DmT, msg, preferred_element_type=jnp.float32))
        hh = jnp.concatenate(outs, axis=0) + bias_ref[li]
        mu = jnp.mean(hh, axis=1, keepdims=True)
        var = jnp.mean((hh - mu) ** 2, axis=1, keepdims=True)
        hh = (hh - mu) * jax.lax.rsqrt(var + 1e-5) * lng_ref[li] + lnb_ref[li]
        hh = jnp.where(hh > 0, hh, jnp.exp(jnp.minimum(hh, 0.0)) - 1.0)
        h = h + hh

    hat = jnp.concatenate([tok, h], axis=1)
    z = jnp.dot(hat, w1_ref[...], preferred_element_type=jnp.float32) + b1_ref[...]
    z = jnp.maximum(z, 0.0)
    logit = jnp.sum(z * w2_ref[...], axis=1, keepdims=True) + b2_ref[...]
    rel = 1.0 / (1.0 + jnp.exp(-logit))

    wgt = h * rel
    num = jnp.dot(rsel_ref[...], wgt, preferred_element_type=jnp.float32)
    den_r = jnp.dot(rsel_ref[...], rel, preferred_element_type=jnp.float32)
    fsc = num / (den_r + 1e-8)

    hat_ref[0] = hat
    rel_ref[0] = rel
    fsc_ref[0] = fsc


def kernel(x, edge_index, edge_attr, params):
    B, N = x.shape
    E = edge_attr.shape[0]
    n_layers = len(params['layers'])
    NB = B // BG
    R = BG * N

    f32 = jnp.float32
    src = edge_index[0]
    dst = edge_index[1]
    S = jax.nn.one_hot(src, N, dtype=f32)
    Dm = jax.nn.one_hot(dst, N, dtype=f32)
    DmT = Dm.T
    G4T = jnp.kron(jnp.eye(H, dtype=f32), jnp.ones((1, C), f32))
    Rsel = jnp.kron(jnp.eye(BG, dtype=f32), jnp.ones((1, N), f32))

    xcol = x.astype(jnp.int32).reshape(NB, R, 1)
    opt_t = jnp.transpose(params['opt_emb'], (1, 0, 2))
    opt_rep = jnp.tile(opt_t, (1, BG, 1))
    attr_rep = jnp.tile(params['attr_emb'], (BG, 1))
    prior_rep = jnp.tile(params['prior'], (BG, 1))

    L = params['layers']
    lw_l = jnp.stack([lp['lin_l_w'] for lp in L])
    lb_l = jnp.stack([lp['lin_l_b'].reshape(1, D) for lp in L])
    lw_r = jnp.stack([lp['lin_r_w'] for lp in L])
    lb_r = jnp.stack([lp['lin_r_b'].reshape(1, D) for lp in L])
    att = jnp.stack([lp['att'].reshape(1, D) for lp in L])
    wedge = jnp.stack([lp['lin_edge_w'].reshape(1, D) for lp in L])
    bias = jnp.stack([lp['bias'].reshape(1, D) for lp in L])
    lng = jnp.stack([lp['ln_g'].reshape(1, D) for lp in L])
    lnb = jnp.stack([lp['ln_b'].reshape(1, D) for lp in L])

    w1 = params['rel_w1']
    b1 = params['rel_b1'].reshape(1, D)
    w2 = params['rel_w2'].reshape(1, D)
    b2 = params['rel_b2'].reshape(1, 1)

    def full(a):
        return pl.BlockSpec(a.shape, lambda i: (0,) * a.ndim)

    consts = [edge_attr, S, Dm, DmT, G4T, Rsel, attr_rep, opt_rep, prior_rep,
              lw_l, lb_l, lw_r, lb_r, att, wedge, bias, lng, lnb,
              w1, b1, w2, b2]

    out_shapes = [
        jax.ShapeDtypeStruct((NB, R, 2 * D), f32),
        jax.ShapeDtypeStruct((NB, R, 1), f32),
        jax.ShapeDtypeStruct((NB, BG, D), f32),
    ]
    out_specs = [
        pl.BlockSpec((1, R, 2 * D), lambda i: (i, 0, 0)),
        pl.BlockSpec((1, R, 1), lambda i: (i, 0, 0)),
        pl.BlockSpec((1, BG, D), lambda i: (i, 0, 0)),
    ]

    hat, rel, fsc = pl.pallas_call(
        functools.partial(_body, n_nodes=N, n_layers=n_layers),
        grid=(NB,),
        in_specs=[pl.BlockSpec((1, R, 1), lambda i: (i, 0, 0))] + [full(a) for a in consts],
        out_specs=out_specs,
        out_shape=out_shapes,
    )(xcol, *consts)

    hat_T = hat.reshape(B, N, 2 * D)
    rel_out = rel.reshape(B, N)
    f_scale = fsc.reshape(B, D)
    return (f_scale, rel_out, hat_T)


# speedup vs baseline: 476.9652x; 2.2746x over previous
import functools

import jax
import jax.numpy as jnp
from jax.experimental import pallas as pl

D, H, C, N_OPT = 64, 4, 16, 5
BG = 8


def _body(xw_ref, ea_ref, s_ref, dm_ref, dmt_ref, g2_ref, g2t_ref,
          gm_ref, gmt_ref, attr_ref, opt_ref, prior_ref,
          lw_l_ref, lb_l_ref, lw_r_ref, lb_r_ref, att_ref, wedge_ref,
          bias_ref, lng_ref, lnb_ref,
          w1_ref, b1_ref, w2_ref, b2_ref,
          hat_ref, rel_ref, fsc_ref, *, n_layers):
    xw = xw_ref[0]
    ea = ea_ref[...]

    tok = jnp.zeros(attr_ref.shape, jnp.float32)
    for o in range(N_OPT):
        tok = tok + jnp.where(xw == o, opt_ref[o], 0.0)
    tok = (attr_ref[...] + tok) * prior_ref[...]

    S = s_ref[...]
    Dm = dm_ref[...]
    DmT = dmt_ref[...]
    G2 = g2_ref[...]
    G2T = g2t_ref[...]
    Gm = gm_ref[...]
    GmT = gmt_ref[...]

    h = tok
    for li in range(n_layers):
        xlw = jnp.dot(h, lw_l_ref[li], preferred_element_type=jnp.float32) + lb_l_ref[li]
        xrw = jnp.dot(h, lw_r_ref[li], preferred_element_type=jnp.float32) + lb_r_ref[li]
        xls = jnp.dot(S, xlw, preferred_element_type=jnp.float32)
        xrd = jnp.dot(Dm, xrw, preferred_element_type=jnp.float32)
        e = xls + xrd + ea * wedge_ref[li]
        e = jnp.where(e >= 0, e, 0.2 * e)
        ew = e * att_ref[li]
        alpha = jnp.dot(ew, G2, preferred_element_type=jnp.float32)
        amax = jnp.max(alpha, axis=0, keepdims=True)
        ex = jnp.exp(alpha - amax)
        den = jnp.dot(DmT, ex, preferred_element_type=jnp.float32)
        den_e = jnp.dot(Dm, den, preferred_element_type=jnp.float32)
        a = ex / (den_e + 1e-16)
        a_exp = jnp.dot(a, G2T, preferred_element_type=jnp.float32)
        msg = xls * a_exp
        hh = jnp.dot(DmT, msg, preferred_element_type=jnp.float32) + bias_ref[li]
        mu = jnp.dot(jnp.dot(hh, Gm, preferred_element_type=jnp.float32) * (1.0 / D),
                     GmT, preferred_element_type=jnp.float32)
        ctr = hh - mu
        var = jnp.dot(jnp.dot(ctr * ctr, Gm, preferred_element_type=jnp.float32) * (1.0 / D),
                      GmT, preferred_element_type=jnp.float32)
        hh = ctr * jax.lax.rsqrt(var + 1e-5) * lng_ref[li] + lnb_ref[li]
        hh = jnp.where(hh > 0, hh, jnp.exp(jnp.minimum(hh, 0.0)) - 1.0)
        h = h + hh

    hat = jnp.concatenate([tok, h], axis=1)
    z = jnp.dot(hat, w1_ref[...], preferred_element_type=jnp.float32) + b1_ref[...]
    z = jnp.maximum(z, 0.0)
    logit = jnp.dot(z * w2_ref[...], Gm, preferred_element_type=jnp.float32) + b2_ref[...]
    relg = 1.0 / (1.0 + jnp.exp(-logit))
    rel_full = jnp.dot(relg, GmT, preferred_element_type=jnp.float32)

    wgt = h * rel_full
    ones_n = jnp.ones((1, wgt.shape[0]), jnp.float32)
    num = jnp.dot(ones_n, wgt, preferred_element_type=jnp.float32)
    den_r = jnp.dot(ones_n, rel_full, preferred_element_type=jnp.float32)
    fsc = num / (den_r + 1e-8)

    hat_ref[0] = hat
    rel_ref[0] = relg
    fsc_ref[0] = fsc


def kernel(x, edge_index, edge_attr, params):
    B, N = x.shape
    E = edge_attr.shape[0]
    n_layers = len(params['layers'])
    NB = B // BG
    W = BG * D

    f32 = jnp.float32
    src = edge_index[0]
    dst = edge_index[1]
    S = jax.nn.one_hot(src, N, dtype=f32)
    Dm = jax.nn.one_hot(dst, N, dtype=f32)
    DmT = Dm.T
    eyeBG = jnp.eye(BG, dtype=f32)
    G2 = jnp.kron(jnp.eye(BG * H, dtype=f32), jnp.ones((C, 1), f32))
    G2T = G2.T
    Gm = jnp.kron(eyeBG, jnp.ones((D, 1), f32))
    GmT = Gm.T

    xw = (x.astype(jnp.int8)
          .reshape(NB, BG, N).transpose(0, 2, 1))
    xw = jnp.repeat(xw, D, axis=2)
    opt_w = jnp.tile(jnp.transpose(params['opt_emb'], (1, 0, 2)), (1, 1, BG))
    attr_w = jnp.tile(params['attr_emb'], (1, BG))
    prior_w = params['prior']

    L = params['layers']

    def bd(w):
        return jnp.kron(eyeBG, w)

    lw_l = jnp.stack([bd(lp['lin_l_w']) for lp in L])
    lb_l = jnp.stack([jnp.tile(lp['lin_l_b'].reshape(1, D), (1, BG)) for lp in L])
    lw_r = jnp.stack([bd(lp['lin_r_w']) for lp in L])
    lb_r = jnp.stack([jnp.tile(lp['lin_r_b'].reshape(1, D), (1, BG)) for lp in L])
    att = jnp.stack([jnp.tile(lp['att'].reshape(1, D), (1, BG)) for lp in L])
    wedge = jnp.stack([jnp.tile(lp['lin_edge_w'].reshape(1, D), (1, BG)) for lp in L])
    bias = jnp.stack([jnp.tile(lp['bias'].reshape(1, D), (1, BG)) for lp in L])
    lng = jnp.stack([jnp.tile(lp['ln_g'].reshape(1, D), (1, BG)) for lp in L])
    lnb = jnp.stack([jnp.tile(lp['ln_b'].reshape(1, D), (1, BG)) for lp in L])

    w1 = params['rel_w1']
    w1big = jnp.concatenate([bd(w1[:D]), bd(w1[D:])], axis=0)
    b1 = jnp.tile(params['rel_b1'].reshape(1, D), (1, BG))
    w2 = jnp.tile(params['rel_w2'].reshape(1, D), (1, BG))
    b2 = params['rel_b2'].reshape(1, 1)

    def full(a):
        return pl.BlockSpec(a.shape, lambda i: (0,) * a.ndim)

    consts = [edge_attr, S, Dm, DmT, G2, G2T, Gm, GmT, attr_w, opt_w, prior_w,
              lw_l, lb_l, lw_r, lb_r, att, wedge, bias, lng, lnb,
              w1big, b1, w2, b2]

    out_shapes = [
        jax.ShapeDtypeStruct((NB, N, 2 * W), f32),
        jax.ShapeDtypeStruct((NB, N, BG), f32),
        jax.ShapeDtypeStruct((NB, 1, W), f32),
    ]
    out_specs = [
        pl.BlockSpec((1, N, 2 * W), lambda i: (i, 0, 0)),
        pl.BlockSpec((1, N, BG), lambda i: (i, 0, 0)),
        pl.BlockSpec((1, 1, W), lambda i: (i, 0, 0)),
    ]

    hat, rel, fsc = pl.pallas_call(
        functools.partial(_body, n_layers=n_layers),
        grid=(NB,),
        in_specs=[pl.BlockSpec((1, N, W), lambda i: (i, 0, 0))] + [full(a) for a in consts],
        out_specs=out_specs,
        out_shape=out_shapes,
    )(xw, *consts)

    hat5 = hat.reshape(NB, N, 2, BG, D)
    hat_T = hat5.transpose(0, 3, 1, 2, 4).reshape(B, N, 2 * D)
    rel_out = rel.transpose(0, 2, 1).reshape(B, N)
    f_scale = fsc.reshape(B, D)
    return (f_scale, rel_out, hat_T)


# speedup vs baseline: 536.4358x; 1.1247x over previous
import functools

import jax
import jax.numpy as jnp
from jax.experimental import pallas as pl

D, H, C, N_OPT = 64, 4, 16, 5
BG = 8


def _body(xw_ref, ea_ref, s_ref, dm_ref, dmt_ref, g2_ref, g2t_ref,
          gm_ref, gmt_ref, attr_ref, opt_ref, prior_ref,
          lw_l_ref, lb_l_ref, lw_r_ref, lb_r_ref, att_ref, wedge_ref,
          bias_ref, lng_ref, lnb_ref,
          w1_ref, b1_ref, w2_ref, b2_ref,
          hat_ref, rel_ref, fsc_ref, *, n_layers):
    xw = xw_ref[0]
    ea = ea_ref[...]

    tok = jnp.zeros(attr_ref.shape, jnp.float32)
    for o in range(N_OPT):
        tok = tok + jnp.where(xw == o, opt_ref[o], 0.0)
    tok = (attr_ref[...] + tok) * prior_ref[...]

    S = s_ref[...]
    Dm = dm_ref[...]
    DmT = dmt_ref[...]
    G2 = g2_ref[...]
    G2T = g2t_ref[...]
    Gm = gm_ref[...]
    GmT = gmt_ref[...]

    h = tok
    for li in range(n_layers):
        xlw = jnp.dot(h, lw_l_ref[li], preferred_element_type=jnp.float32) + lb_l_ref[li]
        xrw = jnp.dot(h, lw_r_ref[li], preferred_element_type=jnp.float32) + lb_r_ref[li]
        xls = jnp.dot(S, xlw, preferred_element_type=jnp.float32)
        xrd = jnp.dot(Dm, xrw, preferred_element_type=jnp.float32)
        e = xls + xrd + ea * wedge_ref[li]
        e = jnp.where(e >= 0, e, 0.2 * e)
        ew = e * att_ref[li]
        alpha = jnp.dot(ew, G2, preferred_element_type=jnp.float32)
        amax = jnp.max(alpha, axis=0, keepdims=True)
        ex = jnp.exp(alpha - amax)
        den = jnp.dot(DmT, ex, preferred_element_type=jnp.float32)
        den_e = jnp.dot(Dm, den, preferred_element_type=jnp.float32)
        a = ex / (den_e + 1e-16)
        a_exp = jnp.dot(a, G2T, preferred_element_type=jnp.float32)
        msg = xls * a_exp
        hh = jnp.dot(DmT, msg, preferred_element_type=jnp.float32) + bias_ref[li]
        mu = jnp.dot(jnp.dot(hh, Gm, preferred_element_type=jnp.float32) * (1.0 / D),
                     GmT, preferred_element_type=jnp.float32)
        ctr = hh - mu
        var = jnp.dot(jnp.dot(ctr * ctr, Gm, preferred_element_type=jnp.float32) * (1.0 / D),
                      GmT, preferred_element_type=jnp.float32)
        hh = ctr * jax.lax.rsqrt(var + 1e-5) * lng_ref[li] + lnb_ref[li]
        hh = jnp.where(hh > 0, hh, jnp.exp(jnp.minimum(hh, 0.0)) - 1.0)
        h = h + hh

    hatw = jnp.concatenate([tok, h], axis=1)
    z = jnp.dot(hatw, w1_ref[...], preferred_element_type=jnp.float32) + b1_ref[...]
    z = jnp.maximum(z, 0.0)
    logit = jnp.dot(z * w2_ref[...], Gm, preferred_element_type=jnp.float32) + b2_ref[...]
    relg = 1.0 / (1.0 + jnp.exp(-logit))
    rel_full = jnp.dot(relg, GmT, preferred_element_type=jnp.float32)

    wgt = h * rel_full
    ones_n = jnp.ones((1, wgt.shape[0]), jnp.float32)
    num = jnp.dot(ones_n, wgt, preferred_element_type=jnp.float32)
    den_r = jnp.dot(ones_n, rel_full, preferred_element_type=jnp.float32)
    fsc = num / (den_r + 1e-8)

    for g in range(BG):
        hat_ref[g] = jnp.concatenate(
            [tok[:, g * D:(g + 1) * D], h[:, g * D:(g + 1) * D]], axis=1)
    rel_ref[0] = relg
    fsc_ref[0] = fsc


def kernel(x, edge_index, edge_attr, params):
    B, N = x.shape
    E = edge_attr.shape[0]
    n_layers = len(params['layers'])
    NB = B // BG
    W = BG * D

    f32 = jnp.float32
    src = edge_index[0]
    dst = edge_index[1]
    S = jax.nn.one_hot(src, N, dtype=f32)
    Dm = jax.nn.one_hot(dst, N, dtype=f32)
    DmT = Dm.T
    eyeBG = jnp.eye(BG, dtype=f32)
    G2 = jnp.kron(jnp.eye(BG * H, dtype=f32), jnp.ones((C, 1), f32))
    G2T = G2.T
    Gm = jnp.kron(eyeBG, jnp.ones((D, 1), f32))
    GmT = Gm.T

    xw = (x.astype(jnp.int8)
          .reshape(NB, BG, N).transpose(0, 2, 1))
    xw = jnp.repeat(xw, D, axis=2)
    opt_w = jnp.tile(jnp.transpose(params['opt_emb'], (1, 0, 2)), (1, 1, BG))
    attr_w = jnp.tile(params['attr_emb'], (1, BG))
    prior_w = params['prior']

    L = params['layers']

    def bd(w):
        return jnp.kron(eyeBG, w)

    lw_l = jnp.stack([bd(lp['lin_l_w']) for lp in L])
    lb_l = jnp.stack([jnp.tile(lp['lin_l_b'].reshape(1, D), (1, BG)) for lp in L])
    lw_r = jnp.stack([bd(lp['lin_r_w']) for lp in L])
    lb_r = jnp.stack([jnp.tile(lp['lin_r_b'].reshape(1, D), (1, BG)) for lp in L])
    att = jnp.stack([jnp.tile(lp['att'].reshape(1, D), (1, BG)) for lp in L])
    wedge = jnp.stack([jnp.tile(lp['lin_edge_w'].reshape(1, D), (1, BG)) for lp in L])
    bias = jnp.stack([jnp.tile(lp['bias'].reshape(1, D), (1, BG)) for lp in L])
    lng = jnp.stack([jnp.tile(lp['ln_g'].reshape(1, D), (1, BG)) for lp in L])
    lnb = jnp.stack([jnp.tile(lp['ln_b'].reshape(1, D), (1, BG)) for lp in L])

    w1 = params['rel_w1']
    w1big = jnp.concatenate([bd(w1[:D]), bd(w1[D:])], axis=0)
    b1 = jnp.tile(params['rel_b1'].reshape(1, D), (1, BG))
    w2 = jnp.tile(params['rel_w2'].reshape(1, D), (1, BG))
    b2 = params['rel_b2'].reshape(1, 1)

    def full(a):
        return pl.BlockSpec(a.shape, lambda i: (0,) * a.ndim)

    consts = [edge_attr, S, Dm, DmT, G2, G2T, Gm, GmT, attr_w, opt_w, prior_w,
              lw_l, lb_l, lw_r, lb_r, att, wedge, bias, lng, lnb,
              w1big, b1, w2, b2]

    out_shapes = [
        jax.ShapeDtypeStruct((B, N, 2 * D), f32),
        jax.ShapeDtypeStruct((NB, N, BG), f32),
        jax.ShapeDtypeStruct((NB, 1, W), f32),
    ]
    out_specs = [
        pl.BlockSpec((BG, N, 2 * D), lambda i: (i, 0, 0)),
        pl.BlockSpec((1, N, BG), lambda i: (i, 0, 0)),
        pl.BlockSpec((1, 1, W), lambda i: (i, 0, 0)),
    ]

    hat, rel, fsc = pl.pallas_call(
        functools.partial(_body, n_layers=n_layers),
        grid=(NB,),
        in_specs=[pl.BlockSpec((1, N, W), lambda i: (i, 0, 0))] + [full(a) for a in consts],
        out_specs=out_specs,
        out_shape=out_shapes,
    )(xw, *consts)

    hat_T = hat
    rel_out = rel.transpose(0, 2, 1).reshape(B, N)
    f_scale = fsc.reshape(B, D)
    return (f_scale, rel_out, hat_T)


# speedup vs baseline: 537.7279x; 1.0024x over previous
import functools

import jax
import jax.numpy as jnp
from jax.experimental import pallas as pl

D, H, C, N_OPT = 64, 4, 16, 5
BG = 8


def _body(xg_ref, ea_ref, src_ref, dst_ref, dmt_ref, g2a_ref, g2t_ref,
          gm_ref, gmt_ref, attr_ref, opt_ref, prior_ref,
          lw_l_ref, lb_l_ref, lw_r_ref, lb_r_ref, wedge_ref,
          bias_ref, lng_ref, lnb_ref,
          w1_ref, b1_ref, w2_ref, b2_ref,
          hat_ref, rel_ref, fsc_ref, *, n_nodes, n_layers):
    N = n_nodes
    xg = xg_ref[0]
    ea = ea_ref[...]
    E = ea.shape[0]

    Gm = gm_ref[...]
    GmT = gmt_ref[...]

    node_iota = jax.lax.broadcasted_iota(jnp.int32, (E, N), 1)
    S = jnp.where(node_iota == src_ref[...], 1.0, 0.0)
    Dm = jnp.where(node_iota == dst_ref[...], 1.0, 0.0)
    DmT = dmt_ref[...]
    G2T = g2t_ref[...]

    tok = jnp.zeros(attr_ref.shape, jnp.float32)
    for o in range(N_OPT):
        mask = jnp.where(xg == o, 1.0, 0.0)
        mfull = jnp.dot(mask, GmT, preferred_element_type=jnp.float32)
        tok = tok + mfull * opt_ref[o]
    tok = (attr_ref[...] + tok) * prior_ref[...]

    h = tok
    for li in range(n_layers):
        xlw = jnp.dot(h, lw_l_ref[li], preferred_element_type=jnp.float32) + lb_l_ref[li]
        xrw = jnp.dot(h, lw_r_ref[li], preferred_element_type=jnp.float32) + lb_r_ref[li]
        xls = jnp.dot(S, xlw, preferred_element_type=jnp.float32)
        xrd = jnp.dot(Dm, xrw, preferred_element_type=jnp.float32)
        e = xls + xrd + ea * wedge_ref[li]
        e = jnp.where(e >= 0, e, 0.2 * e)
        alpha = jnp.dot(e, g2a_ref[li], preferred_element_type=jnp.float32)
        amax = jnp.max(alpha, axis=0, keepdims=True)
        ex = jnp.exp(alpha - amax)
        den = jnp.dot(DmT, ex, preferred_element_type=jnp.float32)
        den_e = jnp.dot(Dm, den, preferred_element_type=jnp.float32)
        a = ex / (den_e + 1e-16)
        a_exp = jnp.dot(a, G2T, preferred_element_type=jnp.float32)
        msg = xls * a_exp
        hh = jnp.dot(DmT, msg, preferred_element_type=jnp.float32) + bias_ref[li]
        mu = jnp.dot(jnp.dot(hh, Gm, preferred_element_type=jnp.float32) * (1.0 / D),
                     GmT, preferred_element_type=jnp.float32)
        ctr = hh - mu
        var = jnp.dot(jnp.dot(ctr * ctr, Gm, preferred_element_type=jnp.float32) * (1.0 / D),
                      GmT, preferred_element_type=jnp.float32)
        hh = ctr * jax.lax.rsqrt(var + 1e-5) * lng_ref[li] + lnb_ref[li]
        hh = jnp.where(hh > 0, hh, jnp.exp(jnp.minimum(hh, 0.0)) - 1.0)
        h = h + hh

    hatw = jnp.concatenate([tok, h], axis=1)
    z = jnp.dot(hatw, w1_ref[...], preferred_element_type=jnp.float32) + b1_ref[...]
    z = jnp.maximum(z, 0.0)
    logit = jnp.dot(z * w2_ref[...], Gm, preferred_element_type=jnp.float32) + b2_ref[...]
    relg = 1.0 / (1.0 + jnp.exp(-logit))
    rel_full = jnp.dot(relg, GmT, preferred_element_type=jnp.float32)

    wgt = h * rel_full
    ones_n = jnp.ones((1, wgt.shape[0]), jnp.float32)
    num = jnp.dot(ones_n, wgt, preferred_element_type=jnp.float32)
    den_r = jnp.dot(ones_n, rel_full, preferred_element_type=jnp.float32)
    fsc = num / (den_r + 1e-8)

    for g in range(BG):
        hat_ref[g] = jnp.concatenate(
            [tok[:, g * D:(g + 1) * D], h[:, g * D:(g + 1) * D]], axis=1)
    rel_ref[0] = relg
    fsc_ref[0] = fsc


def kernel(x, edge_index, edge_attr, params):
    B, N = x.shape
    E = edge_attr.shape[0]
    n_layers = len(params['layers'])
    NB = B // BG
    W = BG * D

    f32 = jnp.float32
    src2 = edge_index[0].astype(jnp.int32).reshape(E, 1)
    dst2 = edge_index[1].astype(jnp.int32).reshape(E, 1)
    DmT = jax.nn.one_hot(edge_index[1], N, dtype=f32).T
    eyeBG = jnp.eye(BG, dtype=f32)
    G2 = jnp.kron(jnp.eye(BG * H, dtype=f32), jnp.ones((C, 1), f32))
    G2T = G2.T
    Gm = jnp.kron(eyeBG, jnp.ones((D, 1), f32))
    GmT = Gm.T

    xg = (x.astype(jnp.int32)
          .reshape(NB, BG, N).transpose(0, 2, 1))
    opt_w = jnp.tile(jnp.transpose(params['opt_emb'], (1, 0, 2)), (1, 1, BG))
    attr_w = jnp.tile(params['attr_emb'], (1, BG))
    prior_w = params['prior']

    L = params['layers']

    def bd(w):
        return jnp.kron(eyeBG, w)

    lw_l = jnp.stack([bd(lp['lin_l_w']) for lp in L])
    lb_l = jnp.stack([jnp.tile(lp['lin_l_b'].reshape(1, D), (1, BG)) for lp in L])
    lw_r = jnp.stack([bd(lp['lin_r_w']) for lp in L])
    lb_r = jnp.stack([jnp.tile(lp['lin_r_b'].reshape(1, D), (1, BG)) for lp in L])
    g2a = jnp.stack([jnp.tile(lp['att'].reshape(1, D), (1, BG)).reshape(W, 1) * G2
                     for lp in L])
    wedge = jnp.stack([jnp.tile(lp['lin_edge_w'].reshape(1, D), (1, BG)) for lp in L])
    bias = jnp.stack([jnp.tile(lp['bias'].reshape(1, D), (1, BG)) for lp in L])
    lng = jnp.stack([jnp.tile(lp['ln_g'].reshape(1, D), (1, BG)) for lp in L])
    lnb = jnp.stack([jnp.tile(lp['ln_b'].reshape(1, D), (1, BG)) for lp in L])

    w1 = params['rel_w1']
    w1big = jnp.concatenate([bd(w1[:D]), bd(w1[D:])], axis=0)
    b1 = jnp.tile(params['rel_b1'].reshape(1, D), (1, BG))
    w2 = jnp.tile(params['rel_w2'].reshape(1, D), (1, BG))
    b2 = params['rel_b2'].reshape(1, 1)

    def full(a):
        return pl.BlockSpec(a.shape, lambda i: (0,) * a.ndim)

    consts = [edge_attr, src2, dst2, DmT, g2a, G2T, Gm, GmT, attr_w, opt_w,
              prior_w, lw_l, lb_l, lw_r, lb_r, wedge, bias, lng, lnb,
              w1big, b1, w2, b2]

    out_shapes = [
        jax.ShapeDtypeStruct((B, N, 2 * D), f32),
        jax.ShapeDtypeStruct((NB, N, BG), f32),
        jax.ShapeDtypeStruct((NB, 1, W), f32),
    ]
    out_specs = [
        pl.BlockSpec((BG, N, 2 * D), lambda i: (i, 0, 0)),
        pl.BlockSpec((1, N, BG), lambda i: (i, 0, 0)),
        pl.BlockSpec((1, 1, W), lambda i: (i, 0, 0)),
    ]

    hat, rel, fsc = pl.pallas_call(
        functools.partial(_body, n_nodes=N, n_layers=n_layers),
        grid=(NB,),
        in_specs=[pl.BlockSpec((1, N, BG), lambda i: (i, 0, 0))] + [full(a) for a in consts],
        out_specs=out_specs,
        out_shape=out_shapes,
    )(xg, *consts)

    hat_T = hat
    rel_out = rel.transpose(0, 2, 1).reshape(B, N)
    f_scale = fsc.reshape(B, D)
    return (f_scale, rel_out, hat_T)


# speedup vs baseline: 552.2507x; 1.0270x over previous
import functools

import jax
import jax.numpy as jnp
from jax.experimental import pallas as pl

D, H, C, N_OPT = 64, 4, 16, 5
BG = 8


def _body(xg_ref, ea_ref, src_ref, dst_ref, dmt_ref, g2a_ref, g2t_ref,
          gm_ref, gmt_ref, attr_ref, opt_ref, prior_ref,
          lw_l_ref, lb_l_ref, lw_r_ref, lb_r_ref, wedge_ref,
          bias_ref, lng_ref, lnb_ref,
          w1_ref, b1_ref, w2_ref, b2_ref,
          hat_ref, rel_ref, fsc_ref, *, n_nodes, n_layers, n_grp):
    N = n_nodes
    ea = ea_ref[...]
    E = ea.shape[0]

    Gm = gm_ref[...]
    GmT = gmt_ref[...]

    node_iota = jax.lax.broadcasted_iota(jnp.int32, (E, N), 1)
    S = jnp.where(node_iota == src_ref[...], 1.0, 0.0)
    Dm = jnp.where(node_iota == dst_ref[...], 1.0, 0.0)
    DmT = dmt_ref[...]
    G2T = g2t_ref[...]

    SDE = jnp.concatenate([S, Dm, ea], axis=1)

    for grp in range(n_grp):
        _one_group(grp, xg_ref, gm_ref, gmt_ref, attr_ref, opt_ref, prior_ref,
                   lw_l_ref, lb_l_ref, lw_r_ref, lb_r_ref, wedge_ref,
                   bias_ref, lng_ref, lnb_ref, w1_ref, b1_ref, w2_ref, b2_ref,
                   hat_ref, rel_ref, fsc_ref, ea, S, Dm, DmT, SDE, G2T, Gm,
                   GmT, g2a_ref, n_layers)


def _one_group(grp, xg_ref, gm_ref, gmt_ref, attr_ref, opt_ref, prior_ref,
               lw_l_ref, lb_l_ref, lw_r_ref, lb_r_ref, wedge_ref,
               bias_ref, lng_ref, lnb_ref, w1_ref, b1_ref, w2_ref, b2_ref,
               hat_ref, rel_ref, fsc_ref, ea, S, Dm, DmT, SDE, G2T, Gm,
               GmT, g2a_ref, n_layers):
    xg = xg_ref[0, :, grp * BG:(grp + 1) * BG]

    tok = jnp.zeros(attr_ref.shape, jnp.float32)
    for o in range(N_OPT):
        mask = jnp.where(xg == o, 1.0, 0.0)
        mfull = jnp.dot(mask, GmT, preferred_element_type=jnp.float32)
        tok = tok + mfull * opt_ref[o]
    tok = (attr_ref[...] + tok) * prior_ref[...]

    h = tok
    for li in range(n_layers):
        xlw = jnp.dot(h, lw_l_ref[li], preferred_element_type=jnp.float32) + lb_l_ref[li]
        xrw = jnp.dot(h, lw_r_ref[li], preferred_element_type=jnp.float32) + lb_r_ref[li]
        xls = jnp.dot(S, xlw, preferred_element_type=jnp.float32)
        z = jnp.concatenate([xlw, xrw, wedge_ref[li]], axis=0)
        e = jnp.dot(SDE, z, preferred_element_type=jnp.float32)
        e = jnp.where(e >= 0, e, 0.2 * e)
        alpha = jnp.dot(e, g2a_ref[li], preferred_element_type=jnp.float32)
        amax = jnp.max(alpha, axis=0, keepdims=True)
        ex = jnp.exp(alpha - amax)
        den = jnp.dot(DmT, ex, preferred_element_type=jnp.float32)
        den_e = jnp.dot(Dm, den, preferred_element_type=jnp.float32)
        a = ex / (den_e + 1e-16)
        a_exp = jnp.dot(a, G2T, preferred_element_type=jnp.float32)
        msg = xls * a_exp
        hh = jnp.dot(DmT, msg, preferred_element_type=jnp.float32) + bias_ref[li]
        mu = jnp.dot(jnp.dot(hh, Gm, preferred_element_type=jnp.float32) * (1.0 / D),
                     GmT, preferred_element_type=jnp.float32)
        ctr = hh - mu
        var = jnp.dot(jnp.dot(ctr * ctr, Gm, preferred_element_type=jnp.float32) * (1.0 / D),
                      GmT, preferred_element_type=jnp.float32)
        hh = ctr * jax.lax.rsqrt(var + 1e-5) * lng_ref[li] + lnb_ref[li]
        hh = jnp.where(hh > 0, hh, jnp.exp(jnp.minimum(hh, 0.0)) - 1.0)
        h = h + hh

    hatw = jnp.concatenate([tok, h], axis=1)
    z = jnp.dot(hatw, w1_ref[...], preferred_element_type=jnp.float32) + b1_ref[...]
    z = jnp.maximum(z, 0.0)
    logit = jnp.dot(z * w2_ref[...], Gm, preferred_element_type=jnp.float32) + b2_ref[...]
    relg = 1.0 / (1.0 + jnp.exp(-logit))
    rel_full = jnp.dot(relg, GmT, preferred_element_type=jnp.float32)

    wgt = h * rel_full
    ones_n = jnp.ones((1, wgt.shape[0]), jnp.float32)
    num = jnp.dot(ones_n, wgt, preferred_element_type=jnp.float32)
    den_r = jnp.dot(ones_n, rel_full, preferred_element_type=jnp.float32)
    fsc = num / (den_r + 1e-8)

    for g in range(BG):
        hat_ref[grp * BG + g] = jnp.concatenate(
            [tok[:, g * D:(g + 1) * D], h[:, g * D:(g + 1) * D]], axis=1)
    rel_ref[0, :, grp * BG:(grp + 1) * BG] = relg
    fsc_ref[0, grp] = fsc[0]


def kernel(x, edge_index, edge_attr, params):
    B, N = x.shape
    E = edge_attr.shape[0]
    n_layers = len(params['layers'])
    NG = 2
    NB = B // (BG * NG)
    W = BG * D

    f32 = jnp.float32
    src2 = edge_index[0].astype(jnp.int32).reshape(E, 1)
    dst2 = edge_index[1].astype(jnp.int32).reshape(E, 1)
    DmT = jax.nn.one_hot(edge_index[1], N, dtype=f32).T
    eyeBG = jnp.eye(BG, dtype=f32)
    G2 = jnp.kron(jnp.eye(BG * H, dtype=f32), jnp.ones((C, 1), f32))
    G2T = G2.T
    Gm = jnp.kron(eyeBG, jnp.ones((D, 1), f32))
    GmT = Gm.T

    xg = (x.astype(jnp.int32)
          .reshape(NB, NG * BG, N).transpose(0, 2, 1))
    opt_w = jnp.tile(jnp.transpose(params['opt_emb'], (1, 0, 2)), (1, 1, BG))
    attr_w = jnp.tile(params['attr_emb'], (1, BG))
    prior_w = params['prior']

    L = params['layers']

    def bd(w):
        return jnp.kron(eyeBG, w)

    lw_l = jnp.stack([bd(lp['lin_l_w']) for lp in L])
    lb_l = jnp.stack([jnp.tile(lp['lin_l_b'].reshape(1, D), (1, BG)) for lp in L])
    lw_r = jnp.stack([bd(lp['lin_r_w']) for lp in L])
    lb_r = jnp.stack([jnp.tile(lp['lin_r_b'].reshape(1, D), (1, BG)) for lp in L])
    g2a = jnp.stack([jnp.tile(lp['att'].reshape(1, D), (1, BG)).reshape(W, 1) * G2
                     for lp in L])
    wedge = jnp.stack([jnp.tile(lp['lin_edge_w'].reshape(1, D), (1, BG)) for lp in L])
    bias = jnp.stack([jnp.tile(lp['bias'].reshape(1, D), (1, BG)) for lp in L])
    lng = jnp.stack([jnp.tile(lp['ln_g'].reshape(1, D), (1, BG)) for lp in L])
    lnb = jnp.stack([jnp.tile(lp['ln_b'].reshape(1, D), (1, BG)) for lp in L])

    w1 = params['rel_w1']
    w1big = jnp.concatenate([bd(w1[:D]), bd(w1[D:])], axis=0)
    b1 = jnp.tile(params['rel_b1'].reshape(1, D), (1, BG))
    w2 = jnp.tile(params['rel_w2'].reshape(1, D), (1, BG))
    b2 = params['rel_b2'].reshape(1, 1)

    def full(a):
        return pl.BlockSpec(a.shape, lambda i: (0,) * a.ndim)

    consts = [edge_attr, src2, dst2, DmT, g2a, G2T, Gm, GmT, attr_w, opt_w,
              prior_w, lw_l, lb_l, lw_r, lb_r, wedge, bias, lng, lnb,
              w1big, b1, w2, b2]

    out_shapes = [
        jax.ShapeDtypeStruct((B, N, 2 * D), f32),
        jax.ShapeDtypeStruct((NB, N, NG * BG), f32),
        jax.ShapeDtypeStruct((NB, NG, W), f32),
    ]
    out_specs = [
        pl.BlockSpec((NG * BG, N, 2 * D), lambda i: (i, 0, 0)),
        pl.BlockSpec((1, N, NG * BG), lambda i: (i, 0, 0)),
        pl.BlockSpec((1, NG, W), lambda i: (i, 0, 0)),
    ]

    hat, rel, fsc = pl.pallas_call(
        functools.partial(_body, n_nodes=N, n_layers=n_layers, n_grp=NG),
        grid=(NB,),
        in_specs=[pl.BlockSpec((1, N, NG * BG), lambda i: (i, 0, 0))] + [full(a) for a in consts],
        out_specs=out_specs,
        out_shape=out_shapes,
    )(xg, *consts)

    hat_T = hat
    rel_out = rel.transpose(0, 2, 1).reshape(B, N)
    f_scale = fsc.reshape(B, D)
    return (f_scale, rel_out, hat_T)


# speedup vs baseline: 555.8932x; 1.0066x over previous
import functools

import jax
import jax.numpy as jnp
from jax.experimental import pallas as pl

D, H, C, N_OPT = 64, 4, 16, 5
BG = 8


def _body(xg_ref, ea_ref, src_ref, dst_ref, dmt_ref, g2a_ref, g2t_ref,
          gm_ref, gmt_ref, attr_ref, opt_ref, prior_ref,
          lw_l_ref, lb_l_ref, lw_r_ref, lb_r_ref, wedge_ref,
          bias_ref, lng_ref, lnb_ref,
          w1_ref, b1_ref, w2_ref, b2_ref,
          hat_ref, rel_ref, fsc_ref, *, n_nodes, n_layers, n_grp):
    N = n_nodes
    ea = ea_ref[...]
    E = ea.shape[0]

    Gm = gm_ref[...]
    GmT = gmt_ref[...]

    node_iota = jax.lax.broadcasted_iota(jnp.int32, (E, N), 1)
    S = jnp.where(node_iota == src_ref[...], 1.0, 0.0)
    Dm = jnp.where(node_iota == dst_ref[...], 1.0, 0.0)
    DmT = dmt_ref[...]
    G2T = g2t_ref[...]

    SDE = jnp.concatenate([S, Dm, ea], axis=1)

    for grp in range(n_grp):
        _one_group(grp, xg_ref, gm_ref, gmt_ref, attr_ref, opt_ref, prior_ref,
                   lw_l_ref, lb_l_ref, lw_r_ref, lb_r_ref, wedge_ref,
                   bias_ref, lng_ref, lnb_ref, w1_ref, b1_ref, w2_ref, b2_ref,
                   hat_ref, rel_ref, fsc_ref, ea, S, Dm, DmT, SDE, G2T, Gm,
                   GmT, g2a_ref, n_layers)


def _one_group(grp, xg_ref, gm_ref, gmt_ref, attr_ref, opt_ref, prior_ref,
               lw_l_ref, lb_l_ref, lw_r_ref, lb_r_ref, wedge_ref,
               bias_ref, lng_ref, lnb_ref, w1_ref, b1_ref, w2_ref, b2_ref,
               hat_ref, rel_ref, fsc_ref, ea, S, Dm, DmT, SDE, G2T, Gm,
               GmT, g2a_ref, n_layers):
    xg = xg_ref[0, :, grp * BG:(grp + 1) * BG]

    tok = jnp.zeros(attr_ref.shape, jnp.float32)
    for o in range(N_OPT):
        mask = jnp.where(xg == o, 1.0, 0.0)
        mfull = jnp.dot(mask, GmT, preferred_element_type=jnp.float32)
        tok = tok + mfull * opt_ref[o]
    tok = (attr_ref[...] + tok) * prior_ref[...]

    h = tok
    for li in range(n_layers):
        xlw = jnp.dot(h, lw_l_ref[li], preferred_element_type=jnp.float32) + lb_l_ref[li]
        xrw = jnp.dot(h, lw_r_ref[li], preferred_element_type=jnp.float32) + lb_r_ref[li]
        xls = jnp.dot(S, xlw, preferred_element_type=jnp.float32)
        z = jnp.concatenate([xlw, xrw, wedge_ref[li]], axis=0)
        e = jnp.dot(SDE, z, preferred_element_type=jnp.float32)
        e = jnp.where(e >= 0, e, 0.2 * e)
        alpha = jnp.dot(e, g2a_ref[li], preferred_element_type=jnp.float32)
        amax = jnp.max(alpha, axis=0, keepdims=True)
        ex = jnp.exp(alpha - amax)
        den = jnp.dot(DmT, ex, preferred_element_type=jnp.float32)
        den_e = jnp.dot(Dm, den, preferred_element_type=jnp.float32)
        a = ex / (den_e + 1e-16)
        a_exp = jnp.dot(a, G2T, preferred_element_type=jnp.float32)
        msg = xls * a_exp
        hh = jnp.dot(DmT, msg, preferred_element_type=jnp.float32) + bias_ref[li]
        mu = jnp.dot(jnp.dot(hh, Gm, preferred_element_type=jnp.float32) * (1.0 / D),
                     GmT, preferred_element_type=jnp.float32)
        ctr = hh - mu
        var = jnp.dot(jnp.dot(ctr * ctr, Gm, preferred_element_type=jnp.float32) * (1.0 / D),
                      GmT, preferred_element_type=jnp.float32)
        hh = ctr * jax.lax.rsqrt(var + 1e-5) * lng_ref[li] + lnb_ref[li]
        hh = jnp.where(hh > 0, hh, jnp.exp(jnp.minimum(hh, 0.0)) - 1.0)
        h = h + hh

    hatw = jnp.concatenate([tok, h], axis=1)
    z = jnp.dot(hatw, w1_ref[...], preferred_element_type=jnp.float32) + b1_ref[...]
    z = jnp.maximum(z, 0.0)
    logit = jnp.dot(z * w2_ref[...], Gm, preferred_element_type=jnp.float32) + b2_ref[...]
    relg = 1.0 / (1.0 + jnp.exp(-logit))
    rel_full = jnp.dot(relg, GmT, preferred_element_type=jnp.float32)

    wgt = h * rel_full
    ones_n = jnp.ones((1, wgt.shape[0]), jnp.float32)
    num = jnp.dot(ones_n, wgt, preferred_element_type=jnp.float32)
    den_r = jnp.dot(ones_n, rel_full, preferred_element_type=jnp.float32)
    fsc = num / (den_r + 1e-8)

    for g in range(BG):
        hat_ref[grp * BG + g] = jnp.concatenate(
            [tok[:, g * D:(g + 1) * D], h[:, g * D:(g + 1) * D]], axis=1)
    rel_ref[0, :, grp * BG:(grp + 1) * BG] = relg
    fsc_ref[0, grp] = fsc[0]


def kernel(x, edge_index, edge_attr, params):
    B, N = x.shape
    E = edge_attr.shape[0]
    n_layers = len(params['layers'])
    NG = 4
    NB = B // (BG * NG)
    W = BG * D

    f32 = jnp.float32
    src2 = edge_index[0].astype(jnp.int32).reshape(E, 1)
    dst2 = edge_index[1].astype(jnp.int32).reshape(E, 1)
    DmT = jax.nn.one_hot(edge_index[1], N, dtype=f32).T
    eyeBG = jnp.eye(BG, dtype=f32)
    G2 = jnp.kron(jnp.eye(BG * H, dtype=f32), jnp.ones((C, 1), f32))
    G2T = G2.T
    Gm = jnp.kron(eyeBG, jnp.ones((D, 1), f32))
    GmT = Gm.T

    xg = (x.astype(jnp.int32)
          .reshape(NB, NG * BG, N).transpose(0, 2, 1))
    opt_w = jnp.tile(jnp.transpose(params['opt_emb'], (1, 0, 2)), (1, 1, BG))
    attr_w = jnp.tile(params['attr_emb'], (1, BG))
    prior_w = params['prior']

    L = params['layers']

    def bd(w):
        return jnp.kron(eyeBG, w)

    lw_l = jnp.stack([bd(lp['lin_l_w']) for lp in L])
    lb_l = jnp.stack([jnp.tile(lp['lin_l_b'].reshape(1, D), (1, BG)) for lp in L])
    lw_r = jnp.stack([bd(lp['lin_r_w']) for lp in L])
    lb_r = jnp.stack([jnp.tile(lp['lin_r_b'].reshape(1, D), (1, BG)) for lp in L])
    g2a = jnp.stack([jnp.tile(lp['att'].reshape(1, D), (1, BG)).reshape(W, 1) * G2
                     for lp in L])
    wedge = jnp.stack([jnp.tile(lp['lin_edge_w'].reshape(1, D), (1, BG)) for lp in L])
    bias = jnp.stack([jnp.tile(lp['bias'].reshape(1, D), (1, BG)) for lp in L])
    lng = jnp.stack([jnp.tile(lp['ln_g'].reshape(1, D), (1, BG)) for lp in L])
    lnb = jnp.stack([jnp.tile(lp['ln_b'].reshape(1, D), (1, BG)) for lp in L])

    w1 = params['rel_w1']
    w1big = jnp.concatenate([bd(w1[:D]), bd(w1[D:])], axis=0)
    b1 = jnp.tile(params['rel_b1'].reshape(1, D), (1, BG))
    w2 = jnp.tile(params['rel_w2'].reshape(1, D), (1, BG))
    b2 = params['rel_b2'].reshape(1, 1)

    def full(a):
        return pl.BlockSpec(a.shape, lambda i: (0,) * a.ndim)

    consts = [edge_attr, src2, dst2, DmT, g2a, G2T, Gm, GmT, attr_w, opt_w,
              prior_w, lw_l, lb_l, lw_r, lb_r, wedge, bias, lng, lnb,
              w1big, b1, w2, b2]

    out_shapes = [
        jax.ShapeDtypeStruct((B, N, 2 * D), f32),
        jax.ShapeDtypeStruct((NB, N, NG * BG), f32),
        jax.ShapeDtypeStruct((NB, NG, W), f32),
    ]
    out_specs = [
        pl.BlockSpec((NG * BG, N, 2 * D), lambda i: (i, 0, 0)),
        pl.BlockSpec((1, N, NG * BG), lambda i: (i, 0, 0)),
        pl.BlockSpec((1, NG, W), lambda i: (i, 0, 0)),
    ]

    hat, rel, fsc = pl.pallas_call(
        functools.partial(_body, n_nodes=N, n_layers=n_layers, n_grp=NG),
        grid=(NB,),
        in_specs=[pl.BlockSpec((1, N, NG * BG), lambda i: (i, 0, 0))] + [full(a) for a in consts],
        out_specs=out_specs,
        out_shape=out_shapes,
    )(xg, *consts)

    hat_T = hat
    rel_out = rel.transpose(0, 2, 1).reshape(B, N)
    f_scale = fsc.reshape(B, D)
    return (f_scale, rel_out, hat_T)
